# baseline (speedup 1.0000x reference)
import functools

import jax
import jax.numpy as jnp
from jax import lax
from jax.experimental import pallas as pl
from jax.experimental.pallas import tpu as pltpu
from jax.experimental.pallas import tpu_sc as plsc

_D = 64
_RANK = 16
_SCALING = 16.0 / 16.0

_NC = 2
_NS = 16
_NW = _NC * _NS

_B = 4096
_L = 50
_TOK = _B * _L
_BPW = _TOK // _NW
_CH = 640
_KROWS = _CH // 128
_NCHUNK = _BPW // _CH

_V = 1000000
_VP = 1 << 20
_HW = _VP // 2
_HA = _VP // 8


def _tw_body(s1_ref, s2_ref, o_ref):
    o_ref[:, 0:_D] = s1_ref[...].T
    o_ref[:, _D:128] = s2_ref[...].T


def _t_w(wt):
    bn = 2048
    nb = _HW // bn
    return pl.pallas_call(
        _tw_body,
        grid=(nb,),
        in_specs=[
            pl.BlockSpec((_D, bn), lambda i: (0, i)),
            pl.BlockSpec((_D, bn), lambda i, _nb=nb: (0, _nb + i)),
        ],
        out_specs=pl.BlockSpec((bn, 128), lambda i: (i, 0)),
        out_shape=jax.ShapeDtypeStruct((_HW, 128), jnp.float32),
    )(wt, wt)


def _ta_body(s0, s1, s2, s3, s4, s5, s6, s7, o_ref):
    srcs = (s0, s1, s2, s3, s4, s5, s6, s7)
    for s in range(8):
        o_ref[:, s * _RANK:(s + 1) * _RANK] = srcs[s][...].T


def _t_a(a):
    bn = 2048
    nb = _HA // bn
    specs = []
    for s in range(8):
        specs.append(
            pl.BlockSpec((_RANK, bn), lambda i, _s=s, _nb=nb: (0, _s * _nb + i))
        )
    return pl.pallas_call(
        _ta_body,
        grid=(nb,),
        in_specs=specs,
        out_specs=pl.BlockSpec((bn, 128), lambda i: (i, 0)),
        out_shape=jax.ShapeDtypeStruct((_HA, 128), jnp.float32),
    )(*([a] * 8))


def _sc_gather_build():
    mesh = plsc.VectorSubcoreMesh(core_axis_name="c", subcore_axis_name="s")

    @functools.partial(
        pl.kernel,
        out_type=(
            jax.ShapeDtypeStruct((_TOK, _D), jnp.float32),
            jax.ShapeDtypeStruct((_TOK, _RANK), jnp.float32),
        ),
        mesh=mesh,
        scratch_types=[
            pltpu.VMEM((_CH,), jnp.int32),
            pltpu.VMEM((_CH,), jnp.int32),
            pltpu.VMEM((_CH,), jnp.int32),
            pltpu.VMEM((_CH,), jnp.int32),
            pltpu.VMEM((_CH, _D), jnp.float32),
            pltpu.VMEM((_CH, _D), jnp.float32),
            pltpu.VMEM((_CH, _RANK), jnp.float32),
            pltpu.VMEM((_CH, _RANK), jnp.float32),
            pltpu.SemaphoreType.DMA,
            pltpu.SemaphoreType.DMA,
        ],
        compiler_params=pltpu.CompilerParams(use_tc_tiling_on_sc=False),
    )
    def sc_gather(
        gw_hbm, ga_hbm, w_hbm, at_hbm, g_out, a_out,
        wi0, wi1, ai0, ai1, g0, g1, a0, a1, gsem, wsem,
    ):
        wid = lax.axis_index("s") * _NC + lax.axis_index("c")
        tok_base = wid * _BPW

        wi_v = (wi0, wi1)
        ai_v = (ai0, ai1)
        g_v = (g0, g1)
        a_v = (a0, a1)
        pending = [None, None]
        for i in range(_NCHUNK):
            b = i & 1
            off = tok_base + i * _CH
            pltpu.sync_copy(gw_hbm.at[pl.ds(off, _CH)], wi_v[b])
            pltpu.sync_copy(ga_hbm.at[pl.ds(off, _CH)], ai_v[b])
            if pending[b] is not None:
                for c in pending[b]:
                    c.wait()
            copies = []
            for j in range(_KROWS):
                wids = wi_v[b].at[pl.ds(j * 128, 128)]
                aids = ai_v[b].at[pl.ds(j * 128, 128)]
                copies.append(
                    pltpu.async_copy(
                        w_hbm.at[wids], g_v[b].at[pl.ds(j * 128, 128)], gsem
                    )
                )
                copies.append(
                    pltpu.async_copy(
                        at_hbm.at[aids], a_v[b].at[pl.ds(j * 128, 128)], gsem
                    )
                )
            for c in copies:
                c.wait()
            pending[b] = [
                pltpu.async_copy(g_v[b], g_out.at[pl.ds(off, _CH)], wsem),
                pltpu.async_copy(a_v[b], a_out.at[pl.ds(off, _CH)], wsem),
            ]
        for b in (0, 1):
            for c in pending[b]:
                c.wait()

    return sc_gather


_sc_gather = _sc_gather_build()


def _combine_body(g_ref, a_ref, b_ref, o_ref):
    lora = lax.dot_general(
        b_ref[...],
        a_ref[...],
        (((1,), (1,)), ((), ())),
        preferred_element_type=jnp.float32,
    )
    o_ref[...] = (g_ref[...].T + lora)[None]


def _tc_combine(g, a, bst):
    bn = 512
    nj = _B // bn
    return pl.pallas_call(
        _combine_body,
        grid=(_L, nj),
        in_specs=[
            pl.BlockSpec((bn, _D), lambda l, j: (l * nj + j, 0)),
            pl.BlockSpec((bn, _RANK), lambda l, j: (l * nj + j, 0)),
            pl.BlockSpec((_D, _RANK), lambda l, j: (0, 0)),
        ],
        out_specs=pl.BlockSpec((1, _D, bn), lambda l, j: (l, 0, j)),
        out_shape=jax.ShapeDtypeStruct((_L, _D, _B), jnp.float32),
    )(g, a, bst)


@jax.jit
def kernel(x, weight, lora_a, lora_b):
    xt = x.T.reshape(_TOK).astype(jnp.int32)
    gw = ((xt & (_HW - 1)) << 1) | lax.shift_right_logical(xt, 19)
    ga = ((xt & (_HA - 1)) << 3) | lax.shift_right_logical(xt, 17)
    wtp = jnp.pad(weight.T, ((0, 0), (0, _VP - _V)))
    ap = jnp.pad(lora_a, ((0, 0), (0, _VP - _V)))
    w2 = _t_w(wtp)
    a2 = _t_a(ap)
    wv = w2.reshape(-1).reshape(_VP, _D)
    atv = a2.reshape(-1).reshape(_VP, _RANK)
    bst = lora_b * _SCALING
    g, a = _sc_gather(gw, ga, wv, atv)
    out = _tc_combine(g, a, bst)
    return out.transpose(2, 0, 1)
